# baseline (speedup 1.0000x reference)
import functools
import jax
import jax.numpy as jnp
from jax import lax
from jax.experimental import pallas as pl
from jax.experimental.pallas import tpu as pltpu
from jax.experimental.pallas import tpu_sc as plsc

E = 8
D = 768
N = 2048
BT = 128
NC = 2
NS = 16
L = 16
NW = NC * NS
TPW = N // NW
S_HALF = 2 * (N // NC) + E * BT
S = NC * S_HALF
NB_HALF = S_HALF // BT
NB = NC * NB_HALF
GW = 128
NEG_INF = float("-inf")


def _gate_body(x_ref, wg_ref, be_ref, i1_ref, i2_ref, g1_ref, g2_ref, b_ref):
    xb = x_ref[...]
    gating = jnp.dot(xb, wg_ref[...], preferred_element_type=jnp.float32)
    iota = lax.broadcasted_iota(jnp.int32, (N, E), 1)
    m1 = jnp.max(gating, axis=1, keepdims=True)
    a1 = jnp.min(jnp.where(gating == m1, iota, E), axis=1, keepdims=True)
    g2d = jnp.where(iota == a1, NEG_INF, gating)
    m2 = jnp.max(g2d, axis=1, keepdims=True)
    a2 = jnp.min(jnp.where(g2d == m2, iota, E), axis=1, keepdims=True)
    t = jnp.exp(m2 - m1)
    w1 = 1.0 / (1.0 + t)
    w2 = 1.0 - w1
    i1_ref[...] = a1
    i2_ref[...] = a2
    g1_ref[...] = w1
    g2_ref[...] = w2
    gates_full = jnp.where(iota == a1, w1, 0.0) + jnp.where(iota == a2, w2, 0.0)
    b_ref[...] = jnp.dot(gates_full, be_ref[...], preferred_element_type=jnp.float32)


_gate_call = pl.pallas_call(
    _gate_body,
    out_shape=[
        jax.ShapeDtypeStruct((N, 1), jnp.int32),
        jax.ShapeDtypeStruct((N, 1), jnp.int32),
        jax.ShapeDtypeStruct((N, 1), jnp.float32),
        jax.ShapeDtypeStruct((N, 1), jnp.float32),
        jax.ShapeDtypeStruct((N, D), jnp.float32),
    ],
)


_sc_mesh = plsc.VectorSubcoreMesh(core_axis_name="c", subcore_axis_name="s")


@functools.partial(
    pl.kernel,
    out_type=[
        jax.ShapeDtypeStruct((S, D), jnp.float32),
        jax.ShapeDtypeStruct((S, GW), jnp.float32),
        jax.ShapeDtypeStruct((N,), jnp.int32),
        jax.ShapeDtypeStruct((N,), jnp.int32),
        jax.ShapeDtypeStruct((2 * 32, 2), jnp.int32),
        jax.ShapeDtypeStruct((NW, L), jnp.int32),
    ],
    mesh=_sc_mesh,
    compiler_params=pltpu.CompilerParams(needs_layout_passes=False),
    scratch_types=[
        pltpu.VMEM((2 * TPW,), jnp.int32),
        pltpu.VMEM((2 * TPW,), jnp.int32),
        pltpu.VMEM((TPW,), jnp.int32),
        pltpu.VMEM((TPW,), jnp.int32),
        pltpu.VMEM((L,), jnp.int32),
        pltpu.VMEM((L,), jnp.int32),
        pltpu.VMEM((NS, L), jnp.int32),
        pltpu.VMEM((TPW, D), jnp.float32),
        pltpu.VMEM((TPW, GW), jnp.float32),
        pltpu.VMEM((TPW, GW), jnp.float32),
        pltpu.VMEM((2 * TPW,), jnp.float32),
        pltpu.VMEM((32, 2), jnp.int32),
        pltpu.VMEM((32,), jnp.int32),
        pltpu.SemaphoreType.DMA,
    ],
)
def _route(x_hbm, i1_hbm, i2_hbm, g1_hbm, g2_hbm,
           xs_hbm, gs_hbm, s1_hbm, s2_hbm, meta_hbm, xch_hbm,
           ids_v, ranks_v, sl1_v, sl2_v, cnt_v, base_v, shr,
           buf, gbuf1, gbuf2, gv, mb2, mbe, sem):
    c = lax.axis_index("c")
    s = lax.axis_index("s")
    wid = c * NS + s
    tok = wid * TPW
    pltpu.sync_copy(i1_hbm.at[pl.ds(tok, TPW)], ids_v.at[pl.ds(0, TPW)])
    pltpu.sync_copy(i2_hbm.at[pl.ds(tok, TPW)], ids_v.at[pl.ds(TPW, TPW)])
    pltpu.sync_copy(g1_hbm.at[pl.ds(tok, TPW)], gv.at[pl.ds(0, TPW)])
    pltpu.sync_copy(g2_hbm.at[pl.ds(tok, TPW)], gv.at[pl.ds(TPW, TPW)])

    lane = lax.broadcasted_iota(jnp.int32, (L,), 0)
    one = jnp.int32(1)
    zero = jnp.int32(0)

    nchunks = 2 * TPW // L
    cnt = [jnp.int32(0)] * E
    for j in range(nchunks):
        v = ids_v[pl.ds(j * L, L)]
        r = jnp.zeros((L,), jnp.int32)
        for e in range(E):
            m = v == e
            mi = jnp.where(m, one, zero)
            cs = plsc.cumsum(mi)
            r = jnp.where(m, cnt[e] + cs - 1, r)
            cnt[e] = cnt[e] + jnp.sum(mi)
        ranks_v[pl.ds(j * L, L)] = r

    cntv = jnp.zeros((L,), jnp.int32)
    for e in range(E):
        cntv = jnp.where(lane == e, cnt[e], cntv)
    cnt_v[...] = cntv

    pltpu.sync_copy(cnt_v, xch_hbm.at[wid])
    plsc.subcore_barrier()
    pltpu.sync_copy(xch_hbm.at[pl.ds(c * NS, NS)], shr)
    tot = jnp.zeros((L,), jnp.int32)
    pre = jnp.zeros((L,), jnp.int32)
    for t in range(NS):
        row = shr[t]
        tot = tot + row
        pre = pre + jnp.where(jnp.int32(t) < s, row, zero)
    pc = ((tot + (BT - 1)) >> 7) << 7
    off = plsc.cumsum(pc) - pc
    base_v[...] = c * S_HALF + off + pre

    for j in range(nchunks):
        v = ids_v[pl.ds(j * L, L)]
        b = plsc.load_gather(base_v, [v])
        sl = b + ranks_v[pl.ds(j * L, L)]
        if j < nchunks // 2:
            sl1_v[pl.ds(j * L, L)] = sl
        else:
            sl2_v[pl.ds((j - nchunks // 2) * L, L)] = sl
    pltpu.sync_copy(sl1_v, s1_hbm.at[pl.ds(tok, TPW)])
    pltpu.sync_copy(sl2_v, s2_hbm.at[pl.ds(tok, TPW)])

    pltpu.sync_copy(x_hbm.at[pl.ds(tok, TPW)], buf)
    pltpu.async_copy(buf, xs_hbm.at[sl1_v], sem).wait()
    pltpu.async_copy(buf, xs_hbm.at[sl2_v], sem).wait()

    for j in range(TPW // L):
        ch1 = gv[pl.ds(j * L, L)]
        ch2 = gv[pl.ds(TPW + j * L, L)]
        for q in range(L):
            r1 = jnp.full((L,), ch1[q], jnp.float32)
            r2 = jnp.full((L,), ch2[q], jnp.float32)
            for w in range(GW // L):
                gbuf1[j * L + q, pl.ds(w * L, L)] = r1
                gbuf2[j * L + q, pl.ds(w * L, L)] = r2
    pltpu.async_copy(gbuf1, gs_hbm.at[sl1_v], sem).wait()
    pltpu.async_copy(gbuf2, gs_hbm.at[sl2_v], sem).wait()

    @pl.when(s == 0)
    def _():
        nb = pc >> 7
        startblk = off >> 7
        totblk = jnp.sum(nb)
        mbe[pl.ds(0, L)] = jnp.zeros((L,), jnp.int32)
        mbe[pl.ds(L, L)] = jnp.zeros((L,), jnp.int32)
        plsc.store_scatter(mbe, [startblk], lane, mask=(nb > 0) & (lane < E))
        c0 = plsc.cummax(mbe[pl.ds(0, L)])
        carry = jnp.max(c0)
        c1 = jnp.maximum(plsc.cummax(mbe[pl.ds(L, L)]), carry)
        for k, ck in enumerate((c0, c1)):
            rows = lane + k * L
            vb = jnp.where(rows < totblk, one, zero)
            plsc.store_scatter(mb2, [rows, jnp.zeros((L,), jnp.int32)], ck)
            plsc.store_scatter(mb2, [rows, jnp.full((L,), one)], vb)
        pltpu.sync_copy(mb2, meta_hbm.at[pl.ds(c * 32, 32)])


def _expert_body(meta_ref, xs_ref, we_ref, gs_ref, ys_ref):
    i = pl.program_id(0)
    r = i + (32 - NB_HALF) * (i // NB_HALF)

    @pl.when(meta_ref[r, 1] == 1)
    def _():
        y = jnp.dot(xs_ref[...], we_ref[0], preferred_element_type=jnp.float32)
        ys_ref[...] = y * gs_ref[:, 0:1]


def _expert_call(meta, xs, We, gs):
    return pl.pallas_call(
        _expert_body,
        grid_spec=pltpu.PrefetchScalarGridSpec(
            num_scalar_prefetch=1,
            grid=(NB,),
            in_specs=[
                pl.BlockSpec((BT, D), lambda i, m: (i, 0)),
                pl.BlockSpec(
                    (1, D, D),
                    lambda i, m: (m[i + (32 - NB_HALF) * (i // NB_HALF), 0], 0, 0),
                ),
                pl.BlockSpec((BT, GW), lambda i, m: (i, 0)),
            ],
            out_specs=pl.BlockSpec((BT, D), lambda i, m: (i, 0)),
        ),
        out_shape=jax.ShapeDtypeStruct((S, D), jnp.float32),
    )(meta, xs, We, gs)


H = TPW // 2


@functools.partial(
    pl.kernel,
    out_type=jax.ShapeDtypeStruct((N, D), jnp.float32),
    mesh=_sc_mesh,
    scratch_types=[
        pltpu.VMEM((TPW,), jnp.int32),
        pltpu.VMEM((TPW,), jnp.int32),
        pltpu.VMEM((H, D), jnp.float32),
        pltpu.VMEM((H, D), jnp.float32),
        pltpu.SemaphoreType.DMA,
    ],
)
def _combine(ys_hbm, s1_hbm, s2_hbm, bias_hbm, out_hbm, sv1, sv2, bufA, bufB, sem):
    c = lax.axis_index("c")
    s = lax.axis_index("s")
    tok = (c * NS + s) * TPW
    pltpu.sync_copy(s1_hbm.at[pl.ds(tok, TPW)], sv1)
    pltpu.sync_copy(s2_hbm.at[pl.ds(tok, TPW)], sv2)

    def add_rows(r, _):
        for k in range(D // L):
            sl = pl.ds(k * L, L)
            bufA[r, sl] = bufA[r, sl] + bufB[r, sl]
        return 0

    for h in range(2):
        pltpu.async_copy(ys_hbm.at[sv1.at[pl.ds(h * H, H)]], bufA, sem).wait()
        pltpu.async_copy(ys_hbm.at[sv2.at[pl.ds(h * H, H)]], bufB, sem).wait()
        lax.fori_loop(0, H, add_rows, 0)
        pltpu.sync_copy(bias_hbm.at[pl.ds(tok + h * H, H)], bufB)
        lax.fori_loop(0, H, add_rows, 0)
        pltpu.sync_copy(bufA, out_hbm.at[pl.ds(tok + h * H, H)])


def kernel(x, W_gate, We, be):
    i1, i2, g1, g2, bias = _gate_call(x, W_gate, be)
    xs, gs, s1, s2, meta, _xch = _route(
        x, i1.reshape(N), i2.reshape(N), g1.reshape(N), g2.reshape(N)
    )
    ys = _expert_call(meta, xs, We, gs)
    return _combine(ys, s1, s2, bias)


# speedup vs baseline: 1.0224x; 1.0224x over previous
import functools
import jax
import jax.numpy as jnp
from jax import lax
from jax.experimental import pallas as pl
from jax.experimental.pallas import tpu as pltpu
from jax.experimental.pallas import tpu_sc as plsc

E = 8
D = 768
N = 2048
BT = 128
NC = 2
NS = 16
L = 16
NW = NC * NS
TPW = N // NW
S_HALF = 2 * (N // NC) + E * BT
S = NC * S_HALF
NB_HALF = S_HALF // BT
NB = NC * NB_HALF
GW = 128
NEG_INF = float("-inf")


def _gate_body(x_ref, wg_ref, be_ref, i1_ref, i2_ref, g1_ref, g2_ref, b_ref,
               base_ref, meta_ref):
    xb = x_ref[...]
    gating = jnp.dot(xb, wg_ref[...], preferred_element_type=jnp.float32)
    iota = lax.broadcasted_iota(jnp.int32, (N, E), 1)
    m1 = jnp.max(gating, axis=1, keepdims=True)
    a1 = jnp.min(jnp.where(gating == m1, iota, E), axis=1, keepdims=True)
    g2d = jnp.where(iota == a1, NEG_INF, gating)
    m2 = jnp.max(g2d, axis=1, keepdims=True)
    a2 = jnp.min(jnp.where(g2d == m2, iota, E), axis=1, keepdims=True)
    t = jnp.exp(m2 - m1)
    w1 = 1.0 / (1.0 + t)
    w2 = 1.0 - w1
    i1_ref[...] = a1
    i2_ref[...] = a2
    g1_ref[...] = w1
    g2_ref[...] = w2
    gates_full = jnp.where(iota == a1, w1, 0.0) + jnp.where(iota == a2, w2, 0.0)
    b_ref[...] = jnp.dot(gates_full, be_ref[...], preferred_element_type=jnp.float32)

    sel = jnp.where(iota == a1, 1.0, 0.0) + jnp.where(iota == a2, 1.0, 0.0)
    cnt = jnp.sum(sel.reshape(NW, TPW, E), axis=1)
    widx = lax.broadcasted_iota(jnp.int32, (NW, NW), 0)
    widy = lax.broadcasted_iota(jnp.int32, (NW, NW), 1)
    same_sc = (widx // NS) == (widy // NS)
    tri = jnp.where(same_sc & (widy < widx), 1.0, 0.0)
    allm = jnp.where(same_sc, 1.0, 0.0)
    pre = jnp.dot(tri, cnt, preferred_element_type=jnp.float32)
    tot = jnp.dot(allm, cnt, preferred_element_type=jnp.float32)
    tot_i = tot.astype(jnp.int32)
    pc = ((tot_i + (BT - 1)) >> 7) << 7
    eiota_x = lax.broadcasted_iota(jnp.int32, (E, E), 0)
    eiota_y = lax.broadcasted_iota(jnp.int32, (E, E), 1)
    excl = jnp.where(eiota_x < eiota_y, 1.0, 0.0)
    off = jnp.dot(pc.astype(jnp.float32), excl,
                  preferred_element_type=jnp.float32).astype(jnp.int32)
    rowc = lax.broadcasted_iota(jnp.int32, (NW, E), 0) // NS
    base = rowc * S_HALF + off + pre.astype(jnp.int32)
    base_ref[...] = jnp.concatenate(
        [base, jnp.zeros((NW, E), jnp.int32)], axis=1)

    r64 = lax.broadcasted_iota(jnp.int32, (64, E), 0)
    jloc = r64 % 32
    pick1 = (r64 // 32) == 1
    sb0 = off[0:1, :] >> 7
    sb1 = off[NS:NS + 1, :] >> 7
    nb0 = pc[0:1, :] >> 7
    nb1 = pc[NS:NS + 1, :] >> 7
    sb = jnp.where(pick1, sb1, sb0)
    nb = jnp.where(pick1, nb1, nb0)
    e64 = lax.broadcasted_iota(jnp.int32, (64, E), 1)
    inrange = (jloc >= sb) & (jloc < sb + nb)
    expert = jnp.max(jnp.where(inrange, e64, 0), axis=1, keepdims=True)
    totblk = jnp.sum(nb, axis=1, keepdims=True)
    validb = jnp.where(jloc[:, 0:1] < totblk, 1, 0)
    meta_ref[...] = jnp.concatenate([expert, validb], axis=1)


_gate_call = pl.pallas_call(
    _gate_body,
    out_shape=[
        jax.ShapeDtypeStruct((N, 1), jnp.int32),
        jax.ShapeDtypeStruct((N, 1), jnp.int32),
        jax.ShapeDtypeStruct((N, 1), jnp.float32),
        jax.ShapeDtypeStruct((N, 1), jnp.float32),
        jax.ShapeDtypeStruct((N, D), jnp.float32),
        jax.ShapeDtypeStruct((NW, L), jnp.int32),
        jax.ShapeDtypeStruct((64, 2), jnp.int32),
    ],
)


_sc_mesh = plsc.VectorSubcoreMesh(core_axis_name="c", subcore_axis_name="s")


@functools.partial(
    pl.kernel,
    out_type=[
        jax.ShapeDtypeStruct((S, D), jnp.float32),
        jax.ShapeDtypeStruct((S, GW), jnp.float32),
        jax.ShapeDtypeStruct((N,), jnp.int32),
        jax.ShapeDtypeStruct((N,), jnp.int32),
    ],
    mesh=_sc_mesh,
    compiler_params=pltpu.CompilerParams(needs_layout_passes=False),
    scratch_types=[
        pltpu.VMEM((2 * TPW,), jnp.int32),
        pltpu.VMEM((2 * TPW,), jnp.int32),
        pltpu.VMEM((TPW,), jnp.int32),
        pltpu.VMEM((TPW,), jnp.int32),
        pltpu.VMEM((L,), jnp.int32),
        pltpu.VMEM((TPW, D), jnp.float32),
        pltpu.VMEM((TPW, GW), jnp.float32),
        pltpu.VMEM((TPW, GW), jnp.float32),
        pltpu.VMEM((2 * TPW,), jnp.float32),
        pltpu.SemaphoreType.DMA,
    ],
)
def _route(x_hbm, i1_hbm, i2_hbm, g1_hbm, g2_hbm, base_hbm,
           xs_hbm, gs_hbm, s1_hbm, s2_hbm,
           ids_v, ranks_v, sl1_v, sl2_v, base_v,
           buf, gbuf1, gbuf2, gv, sem):
    c = lax.axis_index("c")
    s = lax.axis_index("s")
    wid = c * NS + s
    tok = wid * TPW
    pltpu.sync_copy(i1_hbm.at[pl.ds(tok, TPW)], ids_v.at[pl.ds(0, TPW)])
    pltpu.sync_copy(i2_hbm.at[pl.ds(tok, TPW)], ids_v.at[pl.ds(TPW, TPW)])
    pltpu.sync_copy(g1_hbm.at[pl.ds(tok, TPW)], gv.at[pl.ds(0, TPW)])
    pltpu.sync_copy(g2_hbm.at[pl.ds(tok, TPW)], gv.at[pl.ds(TPW, TPW)])
    pltpu.sync_copy(base_hbm.at[wid], base_v)

    one = jnp.int32(1)
    zero = jnp.int32(0)

    nchunks = 2 * TPW // L
    cnt = [jnp.int32(0)] * E
    for j in range(nchunks):
        v = ids_v[pl.ds(j * L, L)]
        r = jnp.zeros((L,), jnp.int32)
        for e in range(E):
            m = v == e
            mi = jnp.where(m, one, zero)
            cs = plsc.cumsum(mi)
            r = jnp.where(m, cnt[e] + cs - 1, r)
            cnt[e] = cnt[e] + jnp.sum(mi)
        ranks_v[pl.ds(j * L, L)] = r

    for j in range(nchunks):
        v = ids_v[pl.ds(j * L, L)]
        b = plsc.load_gather(base_v, [v])
        sl = b + ranks_v[pl.ds(j * L, L)]
        if j < nchunks // 2:
            sl1_v[pl.ds(j * L, L)] = sl
        else:
            sl2_v[pl.ds((j - nchunks // 2) * L, L)] = sl
    pltpu.sync_copy(sl1_v, s1_hbm.at[pl.ds(tok, TPW)])
    pltpu.sync_copy(sl2_v, s2_hbm.at[pl.ds(tok, TPW)])

    pltpu.sync_copy(x_hbm.at[pl.ds(tok, TPW)], buf)
    pltpu.async_copy(buf, xs_hbm.at[sl1_v], sem).wait()
    pltpu.async_copy(buf, xs_hbm.at[sl2_v], sem).wait()

    for j in range(TPW // L):
        ch1 = gv[pl.ds(j * L, L)]
        ch2 = gv[pl.ds(TPW + j * L, L)]
        for q in range(L):
            r1 = jnp.full((L,), ch1[q], jnp.float32)
            r2 = jnp.full((L,), ch2[q], jnp.float32)
            for w in range(GW // L):
                gbuf1[j * L + q, pl.ds(w * L, L)] = r1
                gbuf2[j * L + q, pl.ds(w * L, L)] = r2
    pltpu.async_copy(gbuf1, gs_hbm.at[sl1_v], sem).wait()
    pltpu.async_copy(gbuf2, gs_hbm.at[sl2_v], sem).wait()


def _expert_body(meta_ref, xs_ref, we_ref, gs_ref, ys_ref):
    i = pl.program_id(0)
    r = i + (32 - NB_HALF) * (i // NB_HALF)

    @pl.when(meta_ref[r, 1] == 1)
    def _():
        y = jnp.dot(xs_ref[...], we_ref[0], preferred_element_type=jnp.float32)
        ys_ref[...] = y * gs_ref[:, 0:1]


def _expert_call(meta, xs, We, gs):
    return pl.pallas_call(
        _expert_body,
        grid_spec=pltpu.PrefetchScalarGridSpec(
            num_scalar_prefetch=1,
            grid=(NB,),
            in_specs=[
                pl.BlockSpec((BT, D), lambda i, m: (i, 0)),
                pl.BlockSpec(
                    (1, D, D),
                    lambda i, m: (m[i + (32 - NB_HALF) * (i // NB_HALF), 0], 0, 0),
                ),
                pl.BlockSpec((BT, GW), lambda i, m: (i, 0)),
            ],
            out_specs=pl.BlockSpec((BT, D), lambda i, m: (i, 0)),
        ),
        out_shape=jax.ShapeDtypeStruct((S, D), jnp.float32),
    )(meta, xs, We, gs)


H = TPW // 2


@functools.partial(
    pl.kernel,
    out_type=jax.ShapeDtypeStruct((N, D), jnp.float32),
    mesh=_sc_mesh,
    scratch_types=[
        pltpu.VMEM((TPW,), jnp.int32),
        pltpu.VMEM((TPW,), jnp.int32),
        pltpu.VMEM((H, D), jnp.float32),
        pltpu.VMEM((H, D), jnp.float32),
        pltpu.SemaphoreType.DMA,
    ],
)
def _combine(ys_hbm, s1_hbm, s2_hbm, bias_hbm, out_hbm, sv1, sv2, bufA, bufB, sem):
    c = lax.axis_index("c")
    s = lax.axis_index("s")
    tok = (c * NS + s) * TPW
    pltpu.sync_copy(s1_hbm.at[pl.ds(tok, TPW)], sv1)
    pltpu.sync_copy(s2_hbm.at[pl.ds(tok, TPW)], sv2)

    def add_rows(r, _):
        for k in range(D // L):
            sl = pl.ds(k * L, L)
            bufA[r, sl] = bufA[r, sl] + bufB[r, sl]
        return 0

    for h in range(2):
        pltpu.async_copy(ys_hbm.at[sv1.at[pl.ds(h * H, H)]], bufA, sem).wait()
        pltpu.async_copy(ys_hbm.at[sv2.at[pl.ds(h * H, H)]], bufB, sem).wait()
        lax.fori_loop(0, H, add_rows, 0)
        pltpu.sync_copy(bias_hbm.at[pl.ds(tok + h * H, H)], bufB)
        lax.fori_loop(0, H, add_rows, 0)
        pltpu.sync_copy(bufA, out_hbm.at[pl.ds(tok + h * H, H)])


def kernel(x, W_gate, We, be):
    i1, i2, g1, g2, bias, base, meta = _gate_call(x, W_gate, be)
    xs, gs, s1, s2 = _route(
        x, i1.reshape(N), i2.reshape(N), g1.reshape(N), g2.reshape(N), base
    )
    ys = _expert_call(meta, xs, We.astype(jnp.bfloat16), gs)
    return _combine(ys, s1, s2, bias)


# speedup vs baseline: 1.2718x; 1.2440x over previous
import functools
import jax
import jax.numpy as jnp
from jax import lax
from jax.experimental import pallas as pl
from jax.experimental.pallas import tpu as pltpu
from jax.experimental.pallas import tpu_sc as plsc

E = 8
D = 768
N = 2048
BT = 128
NC = 2
NS = 16
L = 16
NW = NC * NS
TPW = N // NW
S_HALF = 2 * (N // NC) + E * BT
S = NC * S_HALF
NB_HALF = S_HALF // BT
NB = NC * NB_HALF
GW = 128
NEG_INF = float("-inf")


def _gate_body(x_ref, wg_ref, i1_ref, i2_ref, g1_ref, g2_ref,
               base_ref, meta_ref):
    xb = x_ref[...]
    gating = jnp.dot(xb, wg_ref[...], preferred_element_type=jnp.float32)
    iota = lax.broadcasted_iota(jnp.int32, (N, E), 1)
    m1 = jnp.max(gating, axis=1, keepdims=True)
    a1 = jnp.min(jnp.where(gating == m1, iota, E), axis=1, keepdims=True)
    g2d = jnp.where(iota == a1, NEG_INF, gating)
    m2 = jnp.max(g2d, axis=1, keepdims=True)
    a2 = jnp.min(jnp.where(g2d == m2, iota, E), axis=1, keepdims=True)
    t = jnp.exp(m2 - m1)
    w1 = 1.0 / (1.0 + t)
    w2 = 1.0 - w1
    i1_ref[...] = a1[:, 0]
    i2_ref[...] = a2[:, 0]
    g1_ref[...] = w1[:, 0]
    g2_ref[...] = w2[:, 0]

    sel = jnp.where(iota == a1, 1.0, 0.0) + jnp.where(iota == a2, 1.0, 0.0)
    cnt = jnp.sum(sel.reshape(NW, TPW, E), axis=1)
    widx = lax.broadcasted_iota(jnp.int32, (NW, NW), 0)
    widy = lax.broadcasted_iota(jnp.int32, (NW, NW), 1)
    same_sc = (widx // NS) == (widy // NS)
    tri = jnp.where(same_sc & (widy < widx), 1.0, 0.0)
    allm = jnp.where(same_sc, 1.0, 0.0)
    pre = jnp.dot(tri, cnt, preferred_element_type=jnp.float32)
    tot = jnp.dot(allm, cnt, preferred_element_type=jnp.float32)
    tot_i = tot.astype(jnp.int32)
    pc = ((tot_i + (BT - 1)) >> 7) << 7
    eiota_x = lax.broadcasted_iota(jnp.int32, (E, E), 0)
    eiota_y = lax.broadcasted_iota(jnp.int32, (E, E), 1)
    excl = jnp.where(eiota_x < eiota_y, 1.0, 0.0)
    off = jnp.dot(pc.astype(jnp.float32), excl,
                  preferred_element_type=jnp.float32).astype(jnp.int32)
    rowc = lax.broadcasted_iota(jnp.int32, (NW, E), 0) // NS
    base = rowc * S_HALF + off + pre.astype(jnp.int32)
    base_ref[...] = jnp.concatenate(
        [base, jnp.zeros((NW, E), jnp.int32)], axis=1)

    r64 = lax.broadcasted_iota(jnp.int32, (64, E), 0)
    jloc = r64 % 32
    pick1 = (r64 // 32) == 1
    sb0 = off[0:1, :] >> 7
    sb1 = off[NS:NS + 1, :] >> 7
    nb0 = pc[0:1, :] >> 7
    nb1 = pc[NS:NS + 1, :] >> 7
    sb = jnp.where(pick1, sb1, sb0)
    nb = jnp.where(pick1, nb1, nb0)
    e64 = lax.broadcasted_iota(jnp.int32, (64, E), 1)
    inrange = (jloc >= sb) & (jloc < sb + nb)
    expert = jnp.max(jnp.where(inrange, e64, 0), axis=1, keepdims=True)
    totblk = jnp.sum(nb, axis=1, keepdims=True)
    validb = jnp.where(jloc[:, 0:1] < totblk, 1, 0)
    meta_ref[...] = jnp.concatenate([expert, validb], axis=1)


_gate_call = pl.pallas_call(
    _gate_body,
    out_shape=[
        jax.ShapeDtypeStruct((N,), jnp.int32),
        jax.ShapeDtypeStruct((N,), jnp.int32),
        jax.ShapeDtypeStruct((N,), jnp.float32),
        jax.ShapeDtypeStruct((N,), jnp.float32),
        jax.ShapeDtypeStruct((NW, L), jnp.int32),
        jax.ShapeDtypeStruct((64, 2), jnp.int32),
    ],
)


_sc_mesh = plsc.VectorSubcoreMesh(core_axis_name="c", subcore_axis_name="s")


@functools.partial(
    pl.kernel,
    out_type=[
        jax.ShapeDtypeStruct((S, D), jnp.float32),
        jax.ShapeDtypeStruct((S, GW), jnp.float32),
        jax.ShapeDtypeStruct((N,), jnp.int32),
        jax.ShapeDtypeStruct((N,), jnp.int32),
    ],
    mesh=_sc_mesh,
    compiler_params=pltpu.CompilerParams(needs_layout_passes=False),
    scratch_types=[
        pltpu.VMEM((2 * TPW,), jnp.int32),
        pltpu.VMEM((2 * TPW,), jnp.int32),
        pltpu.VMEM((TPW,), jnp.int32),
        pltpu.VMEM((TPW,), jnp.int32),
        pltpu.VMEM((L,), jnp.int32),
        pltpu.VMEM((TPW, D), jnp.float32),
        pltpu.VMEM((TPW, GW), jnp.float32),
        pltpu.VMEM((TPW, GW), jnp.float32),
        pltpu.VMEM((2 * TPW,), jnp.float32),
        pltpu.SemaphoreType.DMA,
    ],
)
def _route(x_hbm, i1_hbm, i2_hbm, g1_hbm, g2_hbm, base_hbm,
           xs_hbm, gs_hbm, s1_hbm, s2_hbm,
           ids_v, ranks_v, sl1_v, sl2_v, base_v,
           buf, gbuf1, gbuf2, gv, sem):
    c = lax.axis_index("c")
    s = lax.axis_index("s")
    wid = c * NS + s
    tok = wid * TPW
    pltpu.sync_copy(i1_hbm.at[pl.ds(tok, TPW)], ids_v.at[pl.ds(0, TPW)])
    pltpu.sync_copy(i2_hbm.at[pl.ds(tok, TPW)], ids_v.at[pl.ds(TPW, TPW)])
    pltpu.sync_copy(g1_hbm.at[pl.ds(tok, TPW)], gv.at[pl.ds(0, TPW)])
    pltpu.sync_copy(g2_hbm.at[pl.ds(tok, TPW)], gv.at[pl.ds(TPW, TPW)])
    pltpu.sync_copy(base_hbm.at[wid], base_v)

    one = jnp.int32(1)
    zero = jnp.int32(0)

    nchunks = 2 * TPW // L
    cnt = [jnp.int32(0)] * E
    for j in range(nchunks):
        v = ids_v[pl.ds(j * L, L)]
        r = jnp.zeros((L,), jnp.int32)
        for e in range(E):
            m = v == e
            mi = jnp.where(m, one, zero)
            cs = plsc.cumsum(mi)
            r = jnp.where(m, cnt[e] + cs - 1, r)
            cnt[e] = cnt[e] + jnp.sum(mi)
        ranks_v[pl.ds(j * L, L)] = r

    for j in range(nchunks):
        v = ids_v[pl.ds(j * L, L)]
        b = plsc.load_gather(base_v, [v])
        sl = b + ranks_v[pl.ds(j * L, L)]
        if j < nchunks // 2:
            sl1_v[pl.ds(j * L, L)] = sl
        else:
            sl2_v[pl.ds((j - nchunks // 2) * L, L)] = sl
    pltpu.sync_copy(sl1_v, s1_hbm.at[pl.ds(tok, TPW)])
    pltpu.sync_copy(sl2_v, s2_hbm.at[pl.ds(tok, TPW)])

    pltpu.sync_copy(x_hbm.at[pl.ds(tok, TPW)], buf)
    pltpu.async_copy(buf, xs_hbm.at[sl1_v], sem).wait()
    pltpu.async_copy(buf, xs_hbm.at[sl2_v], sem).wait()

    for j in range(TPW // L):
        ch1 = gv[pl.ds(j * L, L)]
        ch2 = gv[pl.ds(TPW + j * L, L)]
        for q in range(L):
            r1 = jnp.full((L,), ch1[q], jnp.float32)
            r2 = jnp.full((L,), ch2[q], jnp.float32)
            for w in range(GW // L):
                gbuf1[j * L + q, pl.ds(w * L, L)] = r1
                gbuf2[j * L + q, pl.ds(w * L, L)] = r2
    pltpu.async_copy(gbuf1, gs_hbm.at[sl1_v], sem).wait()
    pltpu.async_copy(gbuf2, gs_hbm.at[sl2_v], sem).wait()


def _expert_body(meta_ref, xs_ref, we_ref, be_ref, gs_ref, ys_ref):
    i = pl.program_id(0)
    r = i + (32 - NB_HALF) * (i // NB_HALF)

    @pl.when(meta_ref[r, 1] == 1)
    def _():
        e = meta_ref[r, 0]
        y = jnp.dot(xs_ref[...], we_ref[e], preferred_element_type=jnp.float32)
        ys_ref[...] = (y + be_ref[e][None, :]) * gs_ref[:, 0:1]


def _expert_call(meta, xs, We, be, gs):
    def _valid_or0(i, m):
        r = i + (32 - NB_HALF) * (i // NB_HALF)
        return jnp.where(m[r, 1] == 1, i, 0)

    return pl.pallas_call(
        _expert_body,
        grid_spec=pltpu.PrefetchScalarGridSpec(
            num_scalar_prefetch=1,
            grid=(NB,),
            in_specs=[
                pl.BlockSpec((BT, D), lambda i, m: (_valid_or0(i, m), 0)),
                pl.BlockSpec((E, D, D), lambda i, m: (0, 0, 0)),
                pl.BlockSpec((E, D), lambda i, m: (0, 0)),
                pl.BlockSpec((BT, GW), lambda i, m: (_valid_or0(i, m), 0)),
            ],
            out_specs=pl.BlockSpec((BT, D), lambda i, m: (i, 0)),
        ),
        out_shape=jax.ShapeDtypeStruct((S, D), jnp.float32),
    )(meta, xs, We, be, gs)


H = TPW // 2


@functools.partial(
    pl.kernel,
    out_type=jax.ShapeDtypeStruct((N, D), jnp.float32),
    mesh=_sc_mesh,
    scratch_types=[
        pltpu.VMEM((TPW,), jnp.int32),
        pltpu.VMEM((TPW,), jnp.int32),
        pltpu.VMEM((2, H, D), jnp.float32),
        pltpu.VMEM((2, H, D), jnp.float32),
        pltpu.SemaphoreType.DMA,
        pltpu.SemaphoreType.DMA,
    ],
)
def _combine(ys_hbm, s1_hbm, s2_hbm, out_hbm, sv1, sv2, bufA, bufB, semA, semB):
    c = lax.axis_index("c")
    s = lax.axis_index("s")
    tok = (c * NS + s) * TPW
    pltpu.sync_copy(s1_hbm.at[pl.ds(tok, TPW)], sv1)
    pltpu.sync_copy(s2_hbm.at[pl.ds(tok, TPW)], sv2)

    cps = []
    for h in range(2):
        ca = pltpu.async_copy(ys_hbm.at[sv1.at[pl.ds(h * H, H)]], bufA.at[h], semA)
        cb = pltpu.async_copy(ys_hbm.at[sv2.at[pl.ds(h * H, H)]], bufB.at[h], semB)
        cps.append((ca, cb))
    for h in range(2):
        ca, cb = cps[h]
        ca.wait()
        cb.wait()

        def add_rows(r, _):
            for k in range(D // L):
                sl = pl.ds(k * L, L)
                bufA[h, r, sl] = bufA[h, r, sl] + bufB[h, r, sl]
            return 0

        lax.fori_loop(0, H, add_rows, 0)
        pltpu.sync_copy(bufA.at[h], out_hbm.at[pl.ds(tok + h * H, H)])


def kernel(x, W_gate, We, be):
    i1, i2, g1, g2, base, meta = _gate_call(x, W_gate)
    xs, gs, s1, s2 = _route(x, i1, i2, g1, g2, base)
    ys = _expert_call(meta, xs, We, be, gs)
    return _combine(ys, s1, s2)


# speedup vs baseline: 1.3633x; 1.0720x over previous
import functools
import jax
import jax.numpy as jnp
from jax import lax
from jax.experimental import pallas as pl
from jax.experimental.pallas import tpu as pltpu
from jax.experimental.pallas import tpu_sc as plsc

E = 8
D = 768
N = 2048
BT = 256
BSH = 8
NC = 2
NS = 16
L = 16
NW = NC * NS
TPW = N // NW
S_HALF = 2 * (N // NC) + E * BT
S = NC * S_HALF
NB_HALF = S_HALF // BT
NB = NC * NB_HALF
GW = 128
NEG_INF = float("-inf")


def _gate_body(x_ref, wg_ref, i1_ref, i2_ref, g1_ref, g2_ref,
               base_ref, meta_ref):
    xb = x_ref[...]
    gating = jnp.dot(xb, wg_ref[...], preferred_element_type=jnp.float32)
    iota = lax.broadcasted_iota(jnp.int32, (N, E), 1)
    m1 = jnp.max(gating, axis=1, keepdims=True)
    a1 = jnp.min(jnp.where(gating == m1, iota, E), axis=1, keepdims=True)
    g2d = jnp.where(iota == a1, NEG_INF, gating)
    m2 = jnp.max(g2d, axis=1, keepdims=True)
    a2 = jnp.min(jnp.where(g2d == m2, iota, E), axis=1, keepdims=True)
    t = jnp.exp(m2 - m1)
    w1 = 1.0 / (1.0 + t)
    w2 = 1.0 - w1
    i1_ref[...] = a1[:, 0]
    i2_ref[...] = a2[:, 0]
    g1_ref[...] = w1[:, 0]
    g2_ref[...] = w2[:, 0]

    sel = jnp.where(iota == a1, 1.0, 0.0) + jnp.where(iota == a2, 1.0, 0.0)
    cnt = jnp.sum(sel.reshape(NW, TPW, E), axis=1)
    widx = lax.broadcasted_iota(jnp.int32, (NW, NW), 0)
    widy = lax.broadcasted_iota(jnp.int32, (NW, NW), 1)
    same_sc = (widx // NS) == (widy // NS)
    tri = jnp.where(same_sc & (widy < widx), 1.0, 0.0)
    allm = jnp.where(same_sc, 1.0, 0.0)
    pre = jnp.dot(tri, cnt, preferred_element_type=jnp.float32)
    tot = jnp.dot(allm, cnt, preferred_element_type=jnp.float32)
    tot_i = tot.astype(jnp.int32)
    pc = ((tot_i + (BT - 1)) >> BSH) << BSH
    eiota_x = lax.broadcasted_iota(jnp.int32, (E, E), 0)
    eiota_y = lax.broadcasted_iota(jnp.int32, (E, E), 1)
    excl = jnp.where(eiota_x < eiota_y, 1.0, 0.0)
    off = jnp.dot(pc.astype(jnp.float32), excl,
                  preferred_element_type=jnp.float32).astype(jnp.int32)
    rowc = lax.broadcasted_iota(jnp.int32, (NW, E), 0) // NS
    base = rowc * S_HALF + off + pre.astype(jnp.int32)
    base_ref[...] = jnp.concatenate(
        [base, jnp.zeros((NW, E), jnp.int32)], axis=1)

    r64 = lax.broadcasted_iota(jnp.int32, (64, E), 0)
    jloc = r64 % 32
    pick1 = (r64 // 32) == 1
    sb0 = off[0:1, :] >> BSH
    sb1 = off[NS:NS + 1, :] >> BSH
    nb0 = pc[0:1, :] >> BSH
    nb1 = pc[NS:NS + 1, :] >> BSH
    sb = jnp.where(pick1, sb1, sb0)
    nb = jnp.where(pick1, nb1, nb0)
    e64 = lax.broadcasted_iota(jnp.int32, (64, E), 1)
    inrange = (jloc >= sb) & (jloc < sb + nb)
    expert = jnp.max(jnp.where(inrange, e64, 0), axis=1, keepdims=True)
    totblk = jnp.sum(nb, axis=1, keepdims=True)
    validb = jnp.where(jloc[:, 0:1] < totblk, 1, 0)
    meta_ref[...] = jnp.concatenate([expert, validb], axis=1)


_gate_call = pl.pallas_call(
    _gate_body,
    out_shape=[
        jax.ShapeDtypeStruct((N,), jnp.int32),
        jax.ShapeDtypeStruct((N,), jnp.int32),
        jax.ShapeDtypeStruct((N,), jnp.float32),
        jax.ShapeDtypeStruct((N,), jnp.float32),
        jax.ShapeDtypeStruct((NW, L), jnp.int32),
        jax.ShapeDtypeStruct((64, 2), jnp.int32),
    ],
)


_sc_mesh = plsc.VectorSubcoreMesh(core_axis_name="c", subcore_axis_name="s")


@functools.partial(
    pl.kernel,
    out_type=[
        jax.ShapeDtypeStruct((S, D), jnp.float32),
        jax.ShapeDtypeStruct((S, GW), jnp.float32),
        jax.ShapeDtypeStruct((N,), jnp.int32),
        jax.ShapeDtypeStruct((N,), jnp.int32),
    ],
    mesh=_sc_mesh,
    compiler_params=pltpu.CompilerParams(needs_layout_passes=False),
    scratch_types=[
        pltpu.VMEM((2 * TPW,), jnp.int32),
        pltpu.VMEM((2 * TPW,), jnp.int32),
        pltpu.VMEM((TPW,), jnp.int32),
        pltpu.VMEM((TPW,), jnp.int32),
        pltpu.VMEM((L,), jnp.int32),
        pltpu.VMEM((TPW, D), jnp.float32),
        pltpu.VMEM((TPW, GW), jnp.float32),
        pltpu.VMEM((TPW, GW), jnp.float32),
        pltpu.VMEM((2 * TPW,), jnp.float32),
        pltpu.SemaphoreType.DMA,
    ],
)
def _route(x_hbm, i1_hbm, i2_hbm, g1_hbm, g2_hbm, base_hbm,
           xs_hbm, gs_hbm, s1_hbm, s2_hbm,
           ids_v, ranks_v, sl1_v, sl2_v, base_v,
           buf, gbuf1, gbuf2, gv, sem):
    c = lax.axis_index("c")
    s = lax.axis_index("s")
    wid = c * NS + s
    tok = wid * TPW
    pltpu.sync_copy(i1_hbm.at[pl.ds(tok, TPW)], ids_v.at[pl.ds(0, TPW)])
    pltpu.sync_copy(i2_hbm.at[pl.ds(tok, TPW)], ids_v.at[pl.ds(TPW, TPW)])
    pltpu.sync_copy(g1_hbm.at[pl.ds(tok, TPW)], gv.at[pl.ds(0, TPW)])
    pltpu.sync_copy(g2_hbm.at[pl.ds(tok, TPW)], gv.at[pl.ds(TPW, TPW)])
    pltpu.sync_copy(base_hbm.at[wid], base_v)

    one = jnp.int32(1)
    zero = jnp.int32(0)

    nchunks = 2 * TPW // L
    cnt = [jnp.int32(0)] * E
    for j in range(nchunks):
        v = ids_v[pl.ds(j * L, L)]
        r = jnp.zeros((L,), jnp.int32)
        for e in range(E):
            m = v == e
            mi = jnp.where(m, one, zero)
            cs = plsc.cumsum(mi)
            r = jnp.where(m, cnt[e] + cs - 1, r)
            cnt[e] = cnt[e] + jnp.sum(mi)
        ranks_v[pl.ds(j * L, L)] = r

    for j in range(nchunks):
        v = ids_v[pl.ds(j * L, L)]
        b = plsc.load_gather(base_v, [v])
        sl = b + ranks_v[pl.ds(j * L, L)]
        if j < nchunks // 2:
            sl1_v[pl.ds(j * L, L)] = sl
        else:
            sl2_v[pl.ds((j - nchunks // 2) * L, L)] = sl
    pltpu.sync_copy(sl1_v, s1_hbm.at[pl.ds(tok, TPW)])
    pltpu.sync_copy(sl2_v, s2_hbm.at[pl.ds(tok, TPW)])

    pltpu.sync_copy(x_hbm.at[pl.ds(tok, TPW)], buf)
    pltpu.async_copy(buf, xs_hbm.at[sl1_v], sem).wait()
    pltpu.async_copy(buf, xs_hbm.at[sl2_v], sem).wait()

    for j in range(TPW // L):
        ch1 = gv[pl.ds(j * L, L)]
        ch2 = gv[pl.ds(TPW + j * L, L)]
        for q in range(L):
            r1 = jnp.full((L,), ch1[q], jnp.float32)
            r2 = jnp.full((L,), ch2[q], jnp.float32)
            for w in range(GW // L):
                gbuf1[j * L + q, pl.ds(w * L, L)] = r1
                gbuf2[j * L + q, pl.ds(w * L, L)] = r2
    pltpu.async_copy(gbuf1, gs_hbm.at[sl1_v], sem).wait()
    pltpu.async_copy(gbuf2, gs_hbm.at[sl2_v], sem).wait()


def _expert_body(meta_ref, xs_ref, we_ref, be_ref, gs_ref, ys_ref):
    i = pl.program_id(0)
    r = i + (32 - NB_HALF) * (i // NB_HALF)

    @pl.when(meta_ref[r, 1] == 1)
    def _():
        e = meta_ref[r, 0]
        y = jnp.dot(xs_ref[...], we_ref[e], preferred_element_type=jnp.float32)
        ys_ref[...] = (y + be_ref[e][None, :]) * gs_ref[:, 0:1]


def _expert_call(meta, xs, We, be, gs):
    def _valid_or0(i, m):
        r = i + (32 - NB_HALF) * (i // NB_HALF)
        return jnp.where(m[r, 1] == 1, i, 0)

    return pl.pallas_call(
        _expert_body,
        grid_spec=pltpu.PrefetchScalarGridSpec(
            num_scalar_prefetch=1,
            grid=(NB,),
            in_specs=[
                pl.BlockSpec((BT, D), lambda i, m: (_valid_or0(i, m), 0)),
                pl.BlockSpec((E, D, D), lambda i, m: (0, 0, 0)),
                pl.BlockSpec((E, D), lambda i, m: (0, 0)),
                pl.BlockSpec((BT, GW), lambda i, m: (_valid_or0(i, m), 0)),
            ],
            out_specs=pl.BlockSpec((BT, D), lambda i, m: (i, 0)),
        ),
        out_shape=jax.ShapeDtypeStruct((S, D), jnp.float32),
    )(meta, xs, We, be, gs)


H = TPW // 2


@functools.partial(
    pl.kernel,
    out_type=jax.ShapeDtypeStruct((N, D), jnp.float32),
    mesh=_sc_mesh,
    scratch_types=[
        pltpu.VMEM((TPW,), jnp.int32),
        pltpu.VMEM((TPW,), jnp.int32),
        pltpu.VMEM((2, H, D), jnp.float32),
        pltpu.VMEM((2, H, D), jnp.float32),
        pltpu.SemaphoreType.DMA,
        pltpu.SemaphoreType.DMA,
    ],
)
def _combine(ys_hbm, s1_hbm, s2_hbm, out_hbm, sv1, sv2, bufA, bufB, semA, semB):
    c = lax.axis_index("c")
    s = lax.axis_index("s")
    tok = (c * NS + s) * TPW
    pltpu.sync_copy(s1_hbm.at[pl.ds(tok, TPW)], sv1)
    pltpu.sync_copy(s2_hbm.at[pl.ds(tok, TPW)], sv2)

    cps = []
    for h in range(2):
        ca = pltpu.async_copy(ys_hbm.at[sv1.at[pl.ds(h * H, H)]], bufA.at[h], semA)
        cb = pltpu.async_copy(ys_hbm.at[sv2.at[pl.ds(h * H, H)]], bufB.at[h], semB)
        cps.append((ca, cb))
    for h in range(2):
        ca, cb = cps[h]
        ca.wait()
        cb.wait()

        def add_rows(r, _):
            for k in range(D // L):
                sl = pl.ds(k * L, L)
                bufA[h, r, sl] = bufA[h, r, sl] + bufB[h, r, sl]
            return 0

        lax.fori_loop(0, H, add_rows, 0)
        pltpu.sync_copy(bufA.at[h], out_hbm.at[pl.ds(tok + h * H, H)])


def kernel(x, W_gate, We, be):
    i1, i2, g1, g2, base, meta = _gate_call(x, W_gate)
    xs, gs, s1, s2 = _route(x, i1, i2, g1, g2, base)
    ys = _expert_call(meta, xs, We, be, gs)
    return _combine(ys, s1, s2)
